# Initial kernel scaffold; baseline (speedup 1.0000x reference)
import jax
import jax.numpy as jnp
from jax.experimental import pallas as pl


def kernel(x, target, mu, logvar, beta):
    raise NotImplementedError("write your pallas kernel here")


# speedup vs baseline: 6.8612x; 6.8612x over previous
import numpy as np
import jax
import jax.numpy as jnp
from jax.experimental import pallas as pl
from jax.experimental.pallas import tpu as pltpu

SIZE = 32000
PAD = 0
SMOOTH = 0.1
CONF = 1.0 - SMOOTH
EPS = SMOOTH / (SIZE - 2)
C_ROW = float(SMOOTH * np.log(EPS) + CONF * np.log(CONF))
N = 4096
D = 512
BC = 640
NBLK = SIZE // BC


def _body(x_ref, t_ref, mu_ref, lv_ref, beta_ref, rec_ref, klb_ref):
    j = pl.program_id(0)
    t = t_ref[...]
    nonpad = (t != PAD).astype(jnp.float32)
    x = x_ref[...]
    cols = j * BC + jax.lax.broadcasted_iota(jnp.int32, (N, BC), 1)
    w = jnp.where(cols == t, -CONF, -EPS)
    partial = jnp.sum(x * w * nonpad)

    @pl.when(j == 0)
    def _():
        cnt = jnp.sum(nonpad)
        x0 = EPS * jnp.sum(x[:, 0:1] * nonpad)
        rec_ref[0, 0] = cnt * C_ROW + x0
        lv = lv_ref[...]
        s = jnp.sum(1.0 + lv - mu_ref[...] * mu_ref[...] - jnp.exp(lv))
        klb_ref[0, 0] = (-0.5 / (N * D)) * s * beta_ref[0]

    rec_ref[0, 0] += partial


def kernel(x, target, mu, logvar, beta):
    t2 = target.astype(jnp.int32).reshape(N, 1)
    rec, klb = pl.pallas_call(
        _body,
        grid=(NBLK,),
        in_specs=[
            pl.BlockSpec((N, BC), lambda j: (0, j)),
            pl.BlockSpec((N, 1), lambda j: (0, 0)),
            pl.BlockSpec((N, D), lambda j: (0, 0)),
            pl.BlockSpec((N, D), lambda j: (0, 0)),
            pl.BlockSpec(memory_space=pltpu.SMEM),
        ],
        out_specs=[
            pl.BlockSpec(memory_space=pltpu.SMEM),
            pl.BlockSpec(memory_space=pltpu.SMEM),
        ],
        out_shape=[
            jax.ShapeDtypeStruct((1, 1), jnp.float32),
            jax.ShapeDtypeStruct((1, 1), jnp.float32),
        ],
    )(x, t2, mu, logvar, beta)
    return rec[0, 0] / N, klb.reshape(1)


# speedup vs baseline: 7.4378x; 1.0840x over previous
import numpy as np
import jax
import jax.numpy as jnp
from jax.experimental import pallas as pl
from jax.experimental.pallas import tpu as pltpu

SIZE = 32000
PAD = 0
SMOOTH = 0.1
CONF = 1.0 - SMOOTH
EPS = SMOOTH / (SIZE - 2)
C_ROW = float(SMOOTH * np.log(EPS) + CONF * np.log(CONF))
N = 4096
D = 512
BC = 1280
NBLK = SIZE // BC


def _body(x_ref, t_ref, mu_ref, lv_ref, beta_ref, rec_ref, klb_ref):
    j = pl.program_id(0)
    t = t_ref[...]
    nonpad = (t != PAD).astype(jnp.float32)
    x = x_ref[...]
    lanes = jax.lax.broadcasted_iota(jnp.int32, (N, BC), 1)
    tshift = t - j * BC
    w_hit = (-CONF) * nonpad
    w_miss = (-EPS) * nonpad
    w = jnp.where(lanes == tshift, w_hit, w_miss)
    partial = jnp.sum(x * w)

    @pl.when(j == 0)
    def _():
        cnt = jnp.sum(nonpad)
        x0 = EPS * jnp.sum(x[:, 0:1] * nonpad)
        rec_ref[0, 0] = cnt * C_ROW + x0
        lv = lv_ref[...]
        s = jnp.sum(1.0 + lv - mu_ref[...] * mu_ref[...] - jnp.exp(lv))
        klb_ref[0, 0] = (-0.5 / (N * D)) * s * beta_ref[0]

    rec_ref[0, 0] += partial


def kernel(x, target, mu, logvar, beta):
    t2 = target.astype(jnp.int32).reshape(N, 1)
    rec, klb = pl.pallas_call(
        _body,
        grid=(NBLK,),
        in_specs=[
            pl.BlockSpec((N, BC), lambda j: (0, j)),
            pl.BlockSpec((N, 1), lambda j: (0, 0)),
            pl.BlockSpec((N, D), lambda j: (0, 0)),
            pl.BlockSpec((N, D), lambda j: (0, 0)),
            pl.BlockSpec(memory_space=pltpu.SMEM),
        ],
        out_specs=[
            pl.BlockSpec(memory_space=pltpu.SMEM),
            pl.BlockSpec(memory_space=pltpu.SMEM),
        ],
        out_shape=[
            jax.ShapeDtypeStruct((1, 1), jnp.float32),
            jax.ShapeDtypeStruct((1, 1), jnp.float32),
        ],
        compiler_params=pltpu.CompilerParams(
            vmem_limit_bytes=100 * 1024 * 1024,
        ),
    )(x, t2, mu, logvar, beta)
    return rec[0, 0] / N, klb.reshape(1)
